# baseline (speedup 1.0000x reference)
import functools

import jax
import jax.numpy as jnp
from jax import lax
from jax.experimental import pallas as pl
from jax.experimental.pallas import tpu as pltpu
from jax.experimental.pallas import tpu_sc as plsc

B, L, D = 4096, 200, 64
NC, NS = 2, 16
NW = NC * NS
IPW = B // NW
PT_S = 2 * L + 1
OB_S = IPW + 1
PT_N = D * PT_S


def _mo(v):
    return pl.multiple_of(v, 8)


def kernel(x, word_table, pos_table):
    xt = x.T
    pt_tab = jnp.concatenate(
        [jnp.zeros((D, L), jnp.float32), pos_table.T[:, 1:L + 1],
         jnp.zeros((D, 1), jnp.float32)], axis=1
    ).reshape(PT_N)

    mesh = plsc.VectorSubcoreMesh(core_axis_name="c", subcore_axis_name="s")

    @functools.partial(
        pl.kernel,
        out_type=jax.ShapeDtypeStruct((L * (D // 8) * (B // 128), 8, 128),
                                      jnp.float32),
        mesh=mesh,
        compiler_params=pltpu.CompilerParams(
            needs_layout_passes=False, use_tc_tiling_on_sc=False),
        scratch_types=[
            pltpu.VMEM((L, IPW), jnp.int32),
            pltpu.VMEM((IPW,), jnp.int32),
            pltpu.VMEM((4 * IPW, D), jnp.float32),
            pltpu.VMEM((4 * D, OB_S), jnp.float32),
            pltpu.VMEM((PT_N,), jnp.float32),
            pltpu.SemaphoreType.DMA,
            pltpu.SemaphoreType.DMA,
            pltpu.SemaphoreType.DMA,
            pltpu.SemaphoreType.DMA,
            pltpu.SemaphoreType.DMA,
            pltpu.SemaphoreType.DMA,
            pltpu.SemaphoreType.DMA,
            pltpu.SemaphoreType.DMA,
        ],
    )
    def run(xt_hbm, wt_hbm, pt_hbm, out_hbm, idx_v, ls_v, gbuf, obuf, pt_v,
            gsem0, gsem1, gsem2, gsem3, ssem0, ssem1, ssem2, ssem3):
        wid = lax.axis_index("s") * NC + lax.axis_index("c")
        i0 = _mo(wid * IPW)
        pltpu.sync_copy(pt_hbm, pt_v)
        pltpu.sync_copy(xt_hbm.at[:, pl.ds(i0, IPW)], idx_v)

        for gi in range(IPW // 16):
            def cnt_body(j2, acc):
                vals = idx_v[j2, pl.ds(gi * 16, 16)]
                return acc + jnp.where(vals != jnp.int32(0),
                                       jnp.int32(1), jnp.int32(0))
            ls_v[pl.ds(gi * 16, 16)] = lax.fori_loop(
                0, L, cnt_body, jnp.zeros((16,), jnp.int32))

        gsems = (gsem0, gsem1, gsem2, gsem3)
        ssems = (ssem0, ssem1, ssem2, ssem3)
        iota = lax.iota(jnp.int32, 16)

        def gather_desc(j, s):
            return pltpu.make_async_copy(
                wt_hbm.at[idx_v.at[j]],
                gbuf.at[pl.ds(s * IPW, IPW)], gsems[s])

        def store_descs(j, s):
            return tuple(
                pltpu.make_async_copy(
                    obuf.at[pl.ds(s * D + dg * 8, 8), pl.ds(0, IPW)],
                    out_hbm.at[(j * (D // 8) + dg) * (B // 128) + wid],
                    ssems[s])
                for dg in range(D // 8))

        def add_row(j, s):
            for tg in range(IPW // 16):
                lsg = ls_v[pl.ds(tg * 16, 16)] + jnp.int32(j)

                @plsc.parallel_loop(0, 16, unroll=2)
                def _(lane):
                    t = tg * 16 + lane
                    lsp = jnp.zeros((16,), jnp.int32) + t
                    pbase = lsg.at[jnp.zeros((16,), jnp.int32) + lane].get(
                        mode="promise_in_bounds")
                    for r in range(D // 16):
                        wv = gbuf[s * IPW + t, pl.ds(r * 16, 16)]
                        pv = plsc.load_gather(
                            pt_v, [(r * 16 + iota) * PT_S + pbase])
                        plsc.store_scatter(
                            obuf, [r * 16 + iota + jnp.int32(s * D), lsp],
                            wv + pv)

        for s in range(4):
            gather_desc(s, s).start()

        def quad_body(g, carry):
            for s in range(4):
                j = 4 * g + s

                @pl.when(g > 0)
                def _():
                    for d in store_descs(0, s):
                        d.wait()

                gather_desc(j, s).wait()
                add_row(j, s)
                for d in store_descs(j, s):
                    d.start()

                @pl.when(g < L // 4 - 1)
                def _():
                    gather_desc(j + 4, s).start()
            return carry

        lax.fori_loop(0, L // 4, quad_body, 0)
        for s in range(4):
            for d in store_descs(0, s):
                d.wait()

    out = run(xt, word_table, pt_tab)
    out = out.reshape(L, D // 8, B // 128, 8, 128)
    out = out.transpose(0, 1, 3, 2, 4).reshape(L, D, B)
    return out.transpose(2, 0, 1)


# speedup vs baseline: 1.0226x; 1.0226x over previous
import functools

import jax
import jax.numpy as jnp
from jax import lax
from jax.experimental import pallas as pl
from jax.experimental.pallas import tpu as pltpu
from jax.experimental.pallas import tpu_sc as plsc

B, L, D = 4096, 200, 64
NC, NS = 2, 16
NW = NC * NS
IPW = B // NW
PT_S = 2 * L + 1
OB_S = IPW + 1
PT_N = D * PT_S


def _mo(v):
    return pl.multiple_of(v, 8)


def kernel(x, word_table, pos_table):
    xt = x.T
    pt_tab = jnp.concatenate(
        [jnp.zeros((D, L), jnp.float32), pos_table.T[:, 1:L + 1],
         jnp.zeros((D, 1), jnp.float32)], axis=1
    ).reshape(PT_N)

    mesh = plsc.VectorSubcoreMesh(core_axis_name="c", subcore_axis_name="s")

    @functools.partial(
        pl.kernel,
        out_type=jax.ShapeDtypeStruct((L * (D // 8) * (B // 128), 8, 128),
                                      jnp.float32),
        mesh=mesh,
        compiler_params=pltpu.CompilerParams(
            needs_layout_passes=False, use_tc_tiling_on_sc=False),
        scratch_types=[
            pltpu.VMEM((L, IPW), jnp.int32),
            pltpu.VMEM((IPW,), jnp.int32),
            pltpu.VMEM((2 * IPW, D), jnp.float32),
            pltpu.VMEM((2 * D, OB_S), jnp.float32),
            pltpu.VMEM((PT_N,), jnp.float32),
            pltpu.SemaphoreType.DMA,
            pltpu.SemaphoreType.DMA,
            pltpu.SemaphoreType.DMA,
            pltpu.SemaphoreType.DMA,
        ],
    )
    def run(xt_hbm, wt_hbm, pt_hbm, out_hbm, idx_v, ls_v, gbuf, obuf, pt_v,
            gsem0, gsem1, ssem0, ssem1):
        wid = lax.axis_index("s") * NC + lax.axis_index("c")
        i0 = _mo(wid * IPW)
        pltpu.sync_copy(pt_hbm, pt_v)
        pltpu.sync_copy(xt_hbm.at[:, pl.ds(i0, IPW)], idx_v)

        for gi in range(IPW // 16):
            def cnt_body(j2, acc):
                vals = idx_v[j2, pl.ds(gi * 16, 16)]
                return acc + jnp.where(vals != jnp.int32(0),
                                       jnp.int32(1), jnp.int32(0))
            ls_v[pl.ds(gi * 16, 16)] = lax.fori_loop(
                0, L, cnt_body, jnp.zeros((16,), jnp.int32))

        gsems = (gsem0, gsem1)
        ssems = (ssem0, ssem1)
        iota = lax.iota(jnp.int32, 16)

        def gather_desc(j, s):
            return pltpu.make_async_copy(
                wt_hbm.at[idx_v.at[j]],
                gbuf.at[pl.ds(s * IPW, IPW)], gsems[s])

        def store_descs(j, s):
            return tuple(
                pltpu.make_async_copy(
                    obuf.at[pl.ds(s * D + dg * 8, 8), pl.ds(0, IPW)],
                    out_hbm.at[(j * (D // 8) + dg) * (B // 128) + wid],
                    ssems[s])
                for dg in range(D // 8))

        def add_row(j, s):
            for tg in range(IPW // 16):
                lsg = ls_v[pl.ds(tg * 16, 16)] + jnp.int32(j)

                @plsc.parallel_loop(0, 16, unroll=2)
                def _(lane):
                    t = tg * 16 + lane
                    lsp = jnp.zeros((16,), jnp.int32) + t
                    pbase = lsg.at[jnp.zeros((16,), jnp.int32) + lane].get(
                        mode="promise_in_bounds")
                    for r in range(D // 16):
                        wv = gbuf[s * IPW + t, pl.ds(r * 16, 16)]
                        pv = plsc.load_gather(
                            pt_v, [(r * 16 + iota) * PT_S + pbase])
                        plsc.store_scatter(
                            obuf, [r * 16 + iota + jnp.int32(s * D), lsp],
                            wv + pv)

        for s in range(2):
            gather_desc(s, s).start()

        def pair_body(g, carry):
            for s in range(2):
                j = 2 * g + s

                @pl.when(g > 0)
                def _():
                    for d in store_descs(0, s):
                        d.wait()

                gather_desc(j, s).wait()
                add_row(j, s)
                for d in store_descs(j, s):
                    d.start()

                @pl.when(g < L // 2 - 1)
                def _():
                    gather_desc(j + 2, s).start()
            return carry

        lax.fori_loop(0, L // 2, pair_body, 0)
        for s in range(2):
            for d in store_descs(0, s):
                d.wait()

    out = run(xt, word_table, pt_tab)
    out = out.reshape(L, D // 8, B // 128, 8, 128)
    out = out.transpose(0, 1, 3, 2, 4).reshape(L, D, B)
    return out.transpose(2, 0, 1)


# speedup vs baseline: 1.1169x; 1.0922x over previous
import functools

import jax
import jax.numpy as jnp
from jax import lax
from jax.experimental import pallas as pl
from jax.experimental.pallas import tpu as pltpu
from jax.experimental.pallas import tpu_sc as plsc

B, L, D = 4096, 200, 64
NC, NS = 2, 16
NW = NC * NS
IPW = B // NW
PT_S = 2 * L + 1
OB_S = IPW + 1
PT_N = D * PT_S


def _mo(v):
    return pl.multiple_of(v, 8)


def kernel(x, word_table, pos_table):
    xt = x.T
    pt_tab = jnp.concatenate(
        [jnp.zeros((D, L), jnp.float32), pos_table.T[:, 1:L + 1],
         jnp.zeros((D, 1), jnp.float32)], axis=1
    ).reshape(PT_N)

    mesh = plsc.VectorSubcoreMesh(core_axis_name="c", subcore_axis_name="s")

    @functools.partial(
        pl.kernel,
        out_type=jax.ShapeDtypeStruct((L * (D // 8) * (B // 128), 8, 128),
                                      jnp.float32),
        mesh=mesh,
        compiler_params=pltpu.CompilerParams(
            needs_layout_passes=False, use_tc_tiling_on_sc=False),
        scratch_types=[
            pltpu.VMEM((L, IPW), jnp.int32),
            pltpu.VMEM((IPW,), jnp.int32),
            pltpu.VMEM((2 * IPW, D), jnp.float32),
            pltpu.VMEM((2 * D, OB_S), jnp.float32),
            pltpu.VMEM((PT_N,), jnp.float32),
            pltpu.SemaphoreType.DMA,
            pltpu.SemaphoreType.DMA,
            pltpu.SemaphoreType.DMA,
            pltpu.SemaphoreType.DMA,
        ],
    )
    def run(xt_hbm, wt_hbm, pt_hbm, out_hbm, idx_v, ls_v, gbuf, obuf, pt_v,
            gsem0, gsem1, ssem0, ssem1):
        wid = lax.axis_index("s") * NC + lax.axis_index("c")
        i0 = _mo(wid * IPW)
        pltpu.sync_copy(pt_hbm, pt_v)
        pltpu.sync_copy(xt_hbm.at[:, pl.ds(i0, IPW)], idx_v)

        for gi in range(IPW // 16):
            def cnt_body(j2, acc):
                vals = idx_v[j2, pl.ds(gi * 16, 16)]
                return acc + jnp.where(vals != jnp.int32(0),
                                       jnp.int32(1), jnp.int32(0))
            ls_v[pl.ds(gi * 16, 16)] = lax.fori_loop(
                0, L, cnt_body, jnp.zeros((16,), jnp.int32))

        gsems = (gsem0, gsem1)
        ssems = (ssem0, ssem1)
        iota = lax.iota(jnp.int32, 16)

        def gather_desc(j, s):
            return pltpu.make_async_copy(
                wt_hbm.at[idx_v.at[j]],
                gbuf.at[pl.ds(s * IPW, IPW)], gsems[s])

        def store_descs(j, s):
            return tuple(
                pltpu.make_async_copy(
                    obuf.at[pl.ds(s * D + dg * 8, 8), pl.ds(0, IPW)],
                    out_hbm.at[(j * (D // 8) + dg) * (B // 128) + wid],
                    ssems[s])
                for dg in range(D // 8))

        def add_row(j, s):
            def tg_body(tg, c):
                lsg = ls_v[pl.ds(tg * 16, 16)] + jnp.int32(j)

                @plsc.parallel_loop(0, 16, unroll=2)
                def _(lane):
                    t = tg * 16 + lane
                    lsp = jnp.zeros((16,), jnp.int32) + t
                    pbase = lsg.at[jnp.zeros((16,), jnp.int32) + lane].get(
                        mode="promise_in_bounds")
                    for r in range(D // 16):
                        wv = gbuf[s * IPW + t, pl.ds(r * 16, 16)]
                        pv = plsc.load_gather(
                            pt_v, [(r * 16 + iota) * PT_S + pbase])
                        plsc.store_scatter(
                            obuf, [r * 16 + iota + jnp.int32(s * D), lsp],
                            wv + pv)
                return c
            lax.fori_loop(0, IPW // 16, tg_body, 0)

        for s in range(2):
            gather_desc(s, s).start()

        def pair_body(g, carry):
            for s in range(2):
                j = 2 * g + s

                @pl.when(g > 0)
                def _():
                    for d in store_descs(0, s):
                        d.wait()

                gather_desc(j, s).wait()
                add_row(j, s)
                for d in store_descs(j, s):
                    d.start()

                @pl.when(g < L // 2 - 1)
                def _():
                    gather_desc(j + 2, s).start()
            return carry

        lax.fori_loop(0, L // 2, pair_body, 0)
        for s in range(2):
            for d in store_descs(0, s):
                d.wait()

    out = run(xt, word_table, pt_tab)
    out = out.reshape(L, D // 8, B // 128, 8, 128)
    out = out.transpose(0, 1, 3, 2, 4).reshape(L, D, B)
    return out.transpose(2, 0, 1)


# speedup vs baseline: 1.1199x; 1.0027x over previous
import functools

import jax
import jax.numpy as jnp
from jax import lax
from jax.experimental import pallas as pl
from jax.experimental.pallas import tpu as pltpu
from jax.experimental.pallas import tpu_sc as plsc

B, L, D = 4096, 200, 64
NC, NS = 2, 16
NW = NC * NS
IPW = B // NW
PT_S = 2 * L + 1
OB_S = IPW + 1
PT_N = D * PT_S


def _mo(v):
    return pl.multiple_of(v, 8)


def kernel(x, word_table, pos_table):
    xt = x.T
    pt_tab = jnp.concatenate(
        [jnp.zeros((D, L), jnp.float32), pos_table.T[:, 1:L + 1],
         jnp.zeros((D, 1), jnp.float32)], axis=1
    ).reshape(PT_N)

    mesh = plsc.VectorSubcoreMesh(core_axis_name="c", subcore_axis_name="s")

    @functools.partial(
        pl.kernel,
        out_type=jax.ShapeDtypeStruct((L * (D // 8) * (B // 128), 8, 128),
                                      jnp.float32),
        mesh=mesh,
        compiler_params=pltpu.CompilerParams(
            needs_layout_passes=False, use_tc_tiling_on_sc=False),
        scratch_types=[
            pltpu.VMEM((L, IPW), jnp.int32),
            pltpu.VMEM((IPW,), jnp.int32),
            pltpu.VMEM((2 * IPW, D), jnp.float32),
            pltpu.VMEM((2 * D, OB_S), jnp.float32),
            pltpu.VMEM((PT_N,), jnp.float32),
            pltpu.SemaphoreType.DMA,
            pltpu.SemaphoreType.DMA,
            pltpu.SemaphoreType.DMA,
            pltpu.SemaphoreType.DMA,
        ],
    )
    def run(xt_hbm, wt_hbm, pt_hbm, out_hbm, idx_v, ls_v, gbuf, obuf, pt_v,
            gsem0, gsem1, ssem0, ssem1):
        wid = lax.axis_index("s") * NC + lax.axis_index("c")
        i0 = _mo(wid * IPW)
        pltpu.sync_copy(pt_hbm, pt_v)
        pltpu.sync_copy(xt_hbm.at[:, pl.ds(i0, IPW)], idx_v)

        for gi in range(IPW // 16):
            def cnt_body(j2, acc):
                vals = idx_v[j2, pl.ds(gi * 16, 16)]
                return acc + jnp.where(vals != jnp.int32(0),
                                       jnp.int32(1), jnp.int32(0))
            ls_v[pl.ds(gi * 16, 16)] = lax.fori_loop(
                0, L, cnt_body, jnp.zeros((16,), jnp.int32))

        gsems = (gsem0, gsem1)
        ssems = (ssem0, ssem1)
        iota = lax.iota(jnp.int32, 16)

        def gather_desc(j, s):
            return pltpu.make_async_copy(
                wt_hbm.at[idx_v.at[j]],
                gbuf.at[pl.ds(s * IPW, IPW)], gsems[s])

        def store_descs(j, s):
            return tuple(
                pltpu.make_async_copy(
                    obuf.at[pl.ds(s * D + dg * 8, 8), pl.ds(0, IPW)],
                    out_hbm.at[(j * (D // 8) + dg) * (B // 128) + wid],
                    ssems[s])
                for dg in range(D // 8))

        def add_row(j, s):
            def tg_body(tg, c):
                lsg = ls_v[pl.ds(tg * 16, 16)] + jnp.int32(j)

                @plsc.parallel_loop(0, 16, unroll=4)
                def _(lane):
                    t = tg * 16 + lane
                    lsp = jnp.zeros((16,), jnp.int32) + t
                    pbase = lsg.at[jnp.zeros((16,), jnp.int32) + lane].get(
                        mode="promise_in_bounds")
                    for r in range(D // 16):
                        wv = gbuf[s * IPW + t, pl.ds(r * 16, 16)]
                        pv = plsc.load_gather(
                            pt_v, [(r * 16 + iota) * PT_S + pbase])
                        plsc.store_scatter(
                            obuf, [r * 16 + iota + jnp.int32(s * D), lsp],
                            wv + pv)
                return c
            lax.fori_loop(0, IPW // 16, tg_body, 0)

        for s in range(2):
            gather_desc(s, s).start()

        def pair_body(g, carry):
            for s in range(2):
                j = 2 * g + s

                @pl.when(g > 0)
                def _():
                    for d in store_descs(0, s):
                        d.wait()

                gather_desc(j, s).wait()
                add_row(j, s)
                for d in store_descs(j, s):
                    d.start()

                @pl.when(g < L // 2 - 1)
                def _():
                    gather_desc(j + 2, s).start()
            return carry

        lax.fori_loop(0, L // 2, pair_body, 0)
        for s in range(2):
            for d in store_descs(0, s):
                d.wait()

    out = run(xt, word_table, pt_tab)
    out = out.reshape(L, D // 8, B // 128, 8, 128)
    out = out.transpose(0, 1, 3, 2, 4).reshape(L, D, B)
    return out.transpose(2, 0, 1)
